# Initial kernel scaffold; baseline (speedup 1.0000x reference)
import jax
import jax.numpy as jnp
from jax.experimental import pallas as pl


def kernel(x, tables, W, b):
    raise NotImplementedError("write your pallas kernel here")


# speedup vs baseline: 1.8120x; 1.8120x over previous
import functools

import jax
import jax.numpy as jnp
from jax import lax
from jax.experimental import pallas as pl
from jax.experimental.pallas import tpu as pltpu
from jax.experimental.pallas import tpu_sc as plsc

B = 16384
INPUT_DIM = 39
NUM_CAT = 26
VOCAB = 100000
EMB = 32
NUM_NUM = 13

NC = 2
NS = 16
NW = NC * NS
SAMPLES_PER_TILE = B // NW
CHUNK = 64
NUM_CHUNKS = SAMPLES_PER_TILE // CHUNK
GROUPS = CHUNK // 16


def _body(x_hbm, tables_hbm, wb_hbm, out_hbm, xv, idxv, rows, wv, outv, sem):
    wid = lax.axis_index("s") * NC + lax.axis_index("c")
    tile_base = pl.multiple_of(wid * SAMPLES_PER_TILE, SAMPLES_PER_TILE)

    pltpu.sync_copy(wb_hbm, wv)

    iota = lax.iota(jnp.int32, 16)

    def chunk_body(c, carry):
        base = pl.multiple_of(tile_base + c * CHUNK, CHUNK)

        pltpu.sync_copy(x_hbm.at[pl.ds(base, CHUNK)], xv)

        for f in range(NUM_CAT):
            col = jnp.full((16,), NUM_NUM + f, jnp.int32)
            for g in range(GROUPS):
                vals = plsc.load_gather(xv, [g * 16 + iota, col])
                idxv[f, pl.ds(g * 16, 16)] = vals.astype(jnp.int32) + f * VOCAB

        copies = [
            pltpu.async_copy(
                tables_hbm.at[idxv.at[f]],
                rows.at[pl.ds(f * CHUNK, CHUNK)],
                sem,
            )
            for f in range(NUM_CAT)
        ]
        for cp in copies:
            cp.wait()

        accs = [jnp.zeros((16,), jnp.float32) for _ in range(GROUPS)]

        def dot_body(t, accs):
            f = t // EMB
            d = t - f * EMB
            w_t = wv[pl.ds(t, 16)][0]
            col = jnp.full((16,), 0, jnp.int32) + d
            out = []
            for g in range(GROUPS):
                rowsel = f * CHUNK + g * 16 + iota
                vals = plsc.load_gather(rows, [rowsel, col])
                out.append(accs[g] + vals * w_t)
            return tuple(out)

        accs = list(lax.fori_loop(0, NUM_CAT * EMB, dot_body, tuple(accs)))

        wtail = wv[pl.ds(832, 16)]
        for i in range(NUM_NUM):
            col = jnp.full((16,), i, jnp.int32)
            w_i = wtail[i]
            for g in range(GROUPS):
                vals = plsc.load_gather(xv, [g * 16 + iota, col])
                accs[g] = accs[g] + vals * w_i
        bias = wtail[NUM_NUM]
        for g in range(GROUPS):
            accs[g] = accs[g] + bias

        for g in range(GROUPS):
            outv[pl.ds(g * 16, 16)] = accs[g]
        pltpu.sync_copy(outv, out_hbm.at[pl.ds(base, CHUNK)])
        return carry

    lax.fori_loop(0, NUM_CHUNKS, chunk_body, 0)


@jax.jit
def kernel(x, tables, W, b):
    tables_flat = tables.reshape(NUM_CAT * VOCAB, EMB)
    wb = jnp.concatenate([W[0], b, jnp.zeros((2,), jnp.float32)])

    mesh = plsc.VectorSubcoreMesh(
        core_axis_name="c", subcore_axis_name="s", num_cores=NC, num_subcores=NS
    )
    run = pl.kernel(
        _body,
        out_type=jax.ShapeDtypeStruct((B,), jnp.float32),
        mesh=mesh,
        compiler_params=pltpu.CompilerParams(
            needs_layout_passes=False, use_tc_tiling_on_sc=False
        ),
        scratch_types=[
            pltpu.VMEM((CHUNK, INPUT_DIM), jnp.float32),
            pltpu.VMEM((NUM_CAT, CHUNK), jnp.int32),
            pltpu.VMEM((NUM_CAT * CHUNK, EMB), jnp.float32),
            pltpu.VMEM((848,), jnp.float32),
            pltpu.VMEM((CHUNK,), jnp.float32),
            pltpu.SemaphoreType.DMA,
        ],
    )
    out = run(x, tables_flat, wb)
    return out.reshape(B, 1)
